# baseline (speedup 1.0000x reference)
import jax
import jax.numpy as jnp
from jax.experimental import pallas as pl
from jax.experimental.pallas import tpu as pltpu

BSZ, N_HEADS, MAX_SEQ_LEN, HEAD_DIM = 8, 16, 4096, 64
SEQLEN = 16
BH = BSZ * N_HEADS


def _body(pos_ref, c_ref, u_ref, o_ref):
    o_ref[...] = c_ref[...]
    p = pos_ref[0]
    o_ref[0, pl.ds(p, SEQLEN), :] = u_ref[0]


def kernel(cache, update, pos):
    c3 = cache.reshape(BH, MAX_SEQ_LEN, HEAD_DIM)
    u3 = update.reshape(BH, SEQLEN, HEAD_DIM)
    out = pl.pallas_call(
        _body,
        grid_spec=pltpu.PrefetchScalarGridSpec(
            num_scalar_prefetch=1,
            grid=(BH,),
            in_specs=[
                pl.BlockSpec((1, MAX_SEQ_LEN, HEAD_DIM), lambda i, p: (i, 0, 0)),
                pl.BlockSpec((1, SEQLEN, HEAD_DIM), lambda i, p: (i, 0, 0)),
            ],
            out_specs=pl.BlockSpec(
                (1, MAX_SEQ_LEN, HEAD_DIM), lambda i, p: (i, 0, 0)
            ),
        ),
        out_shape=jax.ShapeDtypeStruct((BH, MAX_SEQ_LEN, HEAD_DIM), cache.dtype),
    )(pos, c3, u3)
    return out.reshape(BSZ, N_HEADS, MAX_SEQ_LEN, HEAD_DIM)
